# baseline (speedup 1.0000x reference)
import functools

import jax
import jax.numpy as jnp
from jax import lax
from jax.experimental import pallas as pl
from jax.experimental.pallas import tpu as pltpu
from jax.experimental.pallas import tpu_sc as plsc

N_VOCAB = 1000000
N_EMBED = 64
PAD_W = 128
BATCH = 16384
N_SAMPLES = 5

NC = 2
NS = 16
NW = NC * NS
CHUNK = 64
NBUF = 4

OUT_CH = BATCH // (NW * CHUNK)
NZ_CH = BATCH * N_SAMPLES // (NW * CHUNK)
IN_W = BATCH // NW
WAVE = IN_W // 2

_mesh = plsc.VectorSubcoreMesh(core_axis_name="c", subcore_axis_name="s")


@functools.partial(
    pl.kernel,
    mesh=_mesh,
    compiler_params=pltpu.CompilerParams(needs_layout_passes=False),
    out_type=[
        jax.ShapeDtypeStruct((BATCH, N_EMBED), jnp.float32),
        jax.ShapeDtypeStruct((BATCH, N_EMBED), jnp.float32),
        jax.ShapeDtypeStruct((BATCH * N_SAMPLES, N_EMBED), jnp.float32),
    ],
    scratch_types=[
        pltpu.VMEM((IN_W // 16, 16), jnp.int32),
        pltpu.VMEM((OUT_CH, CHUNK), jnp.int32),
        pltpu.VMEM((NZ_CH, CHUNK), jnp.int32),
        pltpu.VMEM((WAVE, N_EMBED), jnp.float32),
        pltpu.VMEM((NBUF, CHUNK, PAD_W), jnp.float32),
        pltpu.VMEM((2, CHUNK, N_EMBED), jnp.float32),
        pltpu.SemaphoreType.DMA,
        pltpu.SemaphoreType.DMA,
        pltpu.SemaphoreType.DMA,
        pltpu.SemaphoreType.DMA,
        pltpu.SemaphoreType.DMA,
        pltpu.SemaphoreType.DMA,
        pltpu.SemaphoreType.DMA,
        pltpu.SemaphoreType.DMA,
    ],
)
def _gather3(in_tab, pout_tab, idx_in, idx_out, idx_nz,
             o_in, o_out, o_nz,
             wi, wo, wn, inbuf, gbufs, obuf,
             isem, issem, g0, g1, g2, g3, s0, s1):
    gsem = (g0, g1, g2, g3)
    osem = (s0, s1)
    w = lax.axis_index("s") * NC + lax.axis_index("c")
    pltpu.sync_copy(idx_in.at[w], wi)
    pltpu.sync_copy(idx_out.at[w], wo)
    pltpu.sync_copy(idx_nz.at[w], wn)

    def fire_in_wave(wave):
        def group(g, carry):
            wv = wi[wave * (WAVE // 16) + g]
            for m in range(16):
                pltpu.async_copy(in_tab.at[wv[m]],
                                 inbuf.at[g * 16 + m], isem)
            return carry
        lax.fori_loop(0, WAVE // 16, group, 0)

    def drain_in_wave(wave):
        base = w * IN_W + wave * WAVE
        pltpu.make_async_copy(o_in.at[pl.ds(base, WAVE)], inbuf, isem).wait()
        pltpu.make_async_copy(inbuf, o_in.at[pl.ds(base, WAVE)], issem).start()

    def run_task(words, out, nch, wbase):
        def g_desc(slot, j):
            return pltpu.make_async_copy(
                pout_tab.at[words.at[j]], gbufs.at[slot], gsem[slot])

        def o_desc(p, j):
            return pltpu.make_async_copy(
                obuf.at[p], out.at[pl.ds(wbase + j * CHUNK, CHUNK)], osem[p])

        for b in range(NBUF):
            g_desc(b, b).start()

        def body(i, carry):
            for b in range(NBUF):
                j = i * NBUF + b
                p = b % 2
                g_desc(b, j).wait()

                @pl.when(j >= 2)
                def _():
                    o_desc(p, j - 2).wait()

                def compact(r, carry2):
                    for k in range(N_EMBED // 16):
                        obuf[p, r, pl.ds(k * 16, 16)] = (
                            gbufs[b, r, pl.ds(k * 16, 16)])
                    return carry2
                lax.fori_loop(0, CHUNK, compact, 0)

                o_desc(p, j).start()

                @pl.when(j + NBUF < nch)
                def _():
                    g_desc(b, j + NBUF).start()
            return carry

        lax.fori_loop(0, nch // NBUF, body, 0)
        o_desc((nch - 2) % 2, nch - 2).wait()
        o_desc((nch - 1) % 2, nch - 1).wait()

    fire_in_wave(0)
    run_task(wo, o_out, OUT_CH, w * OUT_CH * CHUNK)
    drain_in_wave(0)
    pltpu.make_async_copy(inbuf, o_in.at[pl.ds(w * IN_W, WAVE)], issem).wait()
    fire_in_wave(1)
    run_task(wn, o_nz, NZ_CH, w * NZ_CH * CHUNK)
    drain_in_wave(1)
    pltpu.make_async_copy(
        inbuf, o_in.at[pl.ds(w * IN_W + WAVE, WAVE)], issem).wait()


PAD_BLK = 1000


def _pad_body(x_ref, o_ref):
    o_ref[...] = jnp.pad(x_ref[...], ((0, 0), (0, PAD_W - N_EMBED)))


_pad128 = pl.pallas_call(
    _pad_body,
    grid=(N_VOCAB // PAD_BLK,),
    in_specs=[pl.BlockSpec((PAD_BLK, N_EMBED), lambda i: (i, 0))],
    out_specs=pl.BlockSpec((PAD_BLK, PAD_W), lambda i: (i, 0)),
    out_shape=jax.ShapeDtypeStruct((N_VOCAB, PAD_W), jnp.float32),
)


def kernel(in_embed_weight, out_embed_weight, input_words, output_words, noise_words):
    pout = _pad128(out_embed_weight)
    idx_in = input_words.astype(jnp.int32).reshape(NW, IN_W // 16, 16)
    idx_out = output_words.astype(jnp.int32).reshape(NW, OUT_CH, CHUNK)
    idx_nz = noise_words.astype(jnp.int32).reshape(NW, NZ_CH, CHUNK)
    o_in, o_out, o_nz = _gather3(
        in_embed_weight, pout, idx_in, idx_out, idx_nz)
    return (o_in, o_out, o_nz.reshape(BATCH, N_SAMPLES, N_EMBED))


# speedup vs baseline: 1.2181x; 1.2181x over previous
import functools

import jax
import jax.numpy as jnp
from jax import lax
from jax.experimental import pallas as pl
from jax.experimental.pallas import tpu as pltpu
from jax.experimental.pallas import tpu_sc as plsc

N_VOCAB = 1000000
N_EMBED = 64
PAD_W = 128
BATCH = 16384
N_SAMPLES = 5

NC = 2
NS = 16
NW = NC * NS
CHUNK = 128
NBUF = 2

IN_CH = BATCH // (NW * CHUNK)
NZ_CH = BATCH * N_SAMPLES // (NW * CHUNK)

_mesh = plsc.VectorSubcoreMesh(core_axis_name="c", subcore_axis_name="s")


@functools.partial(
    pl.kernel,
    mesh=_mesh,
    compiler_params=pltpu.CompilerParams(needs_layout_passes=False),
    out_type=[
        jax.ShapeDtypeStruct((BATCH, N_EMBED), jnp.float32),
        jax.ShapeDtypeStruct((BATCH, N_EMBED), jnp.float32),
        jax.ShapeDtypeStruct((BATCH * N_SAMPLES, N_EMBED), jnp.float32),
    ],
    scratch_types=[
        pltpu.VMEM((IN_CH, CHUNK), jnp.int32),
        pltpu.VMEM((IN_CH, CHUNK), jnp.int32),
        pltpu.VMEM((IN_CH, CHUNK), jnp.int32),
        pltpu.VMEM((IN_CH, CHUNK), jnp.int32),
        pltpu.VMEM((NZ_CH, CHUNK), jnp.int32),
        pltpu.VMEM((NZ_CH, CHUNK), jnp.int32),
        pltpu.VMEM((NBUF, CHUNK, PAD_W), jnp.float32),
        pltpu.VMEM((2, CHUNK, N_EMBED), jnp.float32),
        pltpu.SemaphoreType.DMA,
        pltpu.SemaphoreType.DMA,
        pltpu.SemaphoreType.DMA,
        pltpu.SemaphoreType.DMA,
    ],
)
def _gather3(in_pairs, out_pairs, blk_in, off_in, blk_out, off_out,
             blk_nz, off_nz, o_in, o_out, o_nz,
             bi, fi, bo, fo, bn, fn, gbufs, obuf,
             g0, g1, s0, s1):
    gsem = (g0, g1)
    osem = (s0, s1)
    w = lax.axis_index("s") * NC + lax.axis_index("c")
    pltpu.sync_copy(blk_in.at[w], bi)
    pltpu.sync_copy(off_in.at[w], fi)
    pltpu.sync_copy(blk_out.at[w], bo)
    pltpu.sync_copy(off_out.at[w], fo)
    pltpu.sync_copy(blk_nz.at[w], bn)
    pltpu.sync_copy(off_nz.at[w], fn)

    def run_task(tab, blks, offs, out, nch, wbase):
        def g_desc(slot, j):
            return pltpu.make_async_copy(
                tab.at[blks.at[j]], gbufs.at[slot], gsem[slot])

        def o_desc(p, j):
            return pltpu.make_async_copy(
                obuf.at[p], out.at[pl.ds(wbase + j * CHUNK, CHUNK)], osem[p])

        for b in range(min(NBUF, nch)):
            g_desc(b, b).start()

        def body(i, carry):
            for b in range(NBUF):
                j = i * NBUF + b
                p = b % 2
                g_desc(b, j).wait()

                @pl.when(j >= 2)
                def _():
                    o_desc(p, j - 2).wait()

                def pick(g, carry2):
                    rv = offs[j, pl.ds(g * 16, 16)]
                    for m in range(16):
                        r = g * 16 + m
                        off = rv[m]
                        for k in range(N_EMBED // 16):
                            obuf[p, r, pl.ds(k * 16, 16)] = (
                                gbufs[b, r, pl.ds(off + k * 16, 16)])
                    return carry2
                lax.fori_loop(0, CHUNK // 16, pick, 0)

                o_desc(p, j).start()

                @pl.when(j + NBUF < nch)
                def _():
                    g_desc(b, j + NBUF).start()
            return carry

        lax.fori_loop(0, nch // NBUF, body, 0)
        o_desc((nch - 2) % 2, nch - 2).wait()
        o_desc((nch - 1) % 2, nch - 1).wait()

    run_task(in_pairs, bi, fi, o_in, IN_CH, w * IN_CH * CHUNK)
    run_task(out_pairs, bo, fo, o_out, IN_CH, w * IN_CH * CHUNK)
    run_task(out_pairs, bn, fn, o_nz, NZ_CH, w * NZ_CH * CHUNK)


def kernel(in_embed_weight, out_embed_weight, input_words, output_words, noise_words):
    in_pairs = in_embed_weight.reshape(N_VOCAB // 2, PAD_W)
    out_pairs = out_embed_weight.reshape(N_VOCAB // 2, PAD_W)

    def split(words, nch):
        wi = words.astype(jnp.int32)
        blk = (wi >> 1).reshape(NW, nch, CHUNK)
        off = ((wi & 1) * N_EMBED).reshape(NW, nch, CHUNK)
        return blk, off

    blk_in, off_in = split(input_words, IN_CH)
    blk_out, off_out = split(output_words, IN_CH)
    blk_nz, off_nz = split(noise_words, NZ_CH)
    o_in, o_out, o_nz = _gather3(
        in_pairs, out_pairs, blk_in, off_in, blk_out, off_out, blk_nz, off_nz)
    return (o_in, o_out, o_nz.reshape(BATCH, N_SAMPLES, N_EMBED))


# speedup vs baseline: 2.6561x; 2.1805x over previous
import functools

import jax
import jax.numpy as jnp
from jax import lax
from jax.experimental import pallas as pl
from jax.experimental.pallas import tpu as pltpu
from jax.experimental.pallas import tpu_sc as plsc

N_VOCAB = 1000000
N_EMBED = 64
BATCH = 16384
N_SAMPLES = 5

NC = 2
NS = 16
NW = NC * NS
BLK = 8
CHUNK = 128
NBUF = 4

IN_CH = BATCH // (NW * CHUNK)
NZ_CH = BATCH * N_SAMPLES // (NW * CHUNK)

_mesh = plsc.VectorSubcoreMesh(core_axis_name="c", subcore_axis_name="s")


@functools.partial(
    pl.kernel,
    mesh=_mesh,
    compiler_params=pltpu.CompilerParams(needs_layout_passes=False),
    out_type=[
        jax.ShapeDtypeStruct((BATCH, N_EMBED), jnp.float32),
        jax.ShapeDtypeStruct((BATCH, N_EMBED), jnp.float32),
        jax.ShapeDtypeStruct((BATCH * N_SAMPLES, N_EMBED), jnp.float32),
    ],
    scratch_types=[
        pltpu.VMEM((IN_CH, CHUNK), jnp.int32),
        pltpu.VMEM((IN_CH, CHUNK), jnp.int32),
        pltpu.VMEM((IN_CH, CHUNK), jnp.int32),
        pltpu.VMEM((IN_CH, CHUNK), jnp.int32),
        pltpu.VMEM((NZ_CH, CHUNK), jnp.int32),
        pltpu.VMEM((NZ_CH, CHUNK), jnp.int32),
        pltpu.VMEM((NBUF, CHUNK, N_EMBED), jnp.float32),
        pltpu.SemaphoreType.DMA,
        pltpu.SemaphoreType.DMA,
        pltpu.SemaphoreType.DMA,
        pltpu.SemaphoreType.DMA,
        pltpu.SemaphoreType.DMA,
        pltpu.SemaphoreType.DMA,
        pltpu.SemaphoreType.DMA,
        pltpu.SemaphoreType.DMA,
    ],
)
def _gather3(in_tab, out_tab, blk_in, rem_in, blk_out, rem_out, blk_nz, rem_nz,
             o_in, o_out, o_nz,
             bi, ri, bo, ro, bn, rn, bufs, *sems):
    gsem = sems[:NBUF]
    ssem = sems[NBUF:]
    w = lax.axis_index("s") * NC + lax.axis_index("c")
    pltpu.sync_copy(blk_in.at[w], bi)
    pltpu.sync_copy(rem_in.at[w], ri)
    pltpu.sync_copy(blk_out.at[w], bo)
    pltpu.sync_copy(rem_out.at[w], ro)
    pltpu.sync_copy(blk_nz.at[w], bn)
    pltpu.sync_copy(rem_nz.at[w], rn)

    def run_task(tab, blks, rems, out, nch, wbase):
        def issue_rows(slot, j):
            def group(g, carry):
                bv = blks[j, pl.ds(g * 16, 16)]
                rv = rems[j, pl.ds(g * 16, 16)]
                for m in range(16):
                    pltpu.async_copy(tab.at[bv[m], rv[m]],
                                     bufs.at[slot, g * 16 + m], gsem[slot])
                return carry
            lax.fori_loop(0, CHUNK // 16, group, 0)

        def drain_rows(slot, j):
            pltpu.make_async_copy(
                out.at[pl.ds(wbase + j * CHUNK, CHUNK)], bufs.at[slot],
                gsem[slot]).wait()

        def s_desc(slot, j):
            return pltpu.make_async_copy(
                bufs.at[slot], out.at[pl.ds(wbase + j * CHUNK, CHUNK)],
                ssem[slot])

        for b in range(2):
            issue_rows(b, b)

        def body(i, carry):
            for b in range(NBUF):
                j = i * NBUF + b

                @pl.when(j - 2 >= 0)
                def _():
                    s_desc((b + 2) % NBUF, j - 2).wait()

                @pl.when(j + 2 < nch)
                def _():
                    issue_rows((b + 2) % NBUF, j + 2)

                drain_rows(b, j)
                s_desc(b, j).start()
            return carry

        lax.fori_loop(0, nch // NBUF, body, 0)
        s_desc((nch - 2) % NBUF, nch - 2).wait()
        s_desc((nch - 1) % NBUF, nch - 1).wait()

    run_task(in_tab, bi, ri, o_in, IN_CH, w * IN_CH * CHUNK)
    run_task(out_tab, bo, ro, o_out, IN_CH, w * IN_CH * CHUNK)
    run_task(out_tab, bn, rn, o_nz, NZ_CH, w * NZ_CH * CHUNK)


def kernel(in_embed_weight, out_embed_weight, input_words, output_words, noise_words):
    tab_in = in_embed_weight.reshape(N_VOCAB // BLK, BLK, N_EMBED)
    tab_out = out_embed_weight.reshape(N_VOCAB // BLK, BLK, N_EMBED)

    def split(words, nch):
        wi = words.astype(jnp.int32)
        blk = (wi >> 3).reshape(NW, nch, CHUNK)
        rem = (wi & 7).reshape(NW, nch, CHUNK)
        return blk, rem

    blk_in, rem_in = split(input_words, IN_CH)
    blk_out, rem_out = split(output_words, IN_CH)
    blk_nz, rem_nz = split(noise_words, NZ_CH)
    o_in, o_out, o_nz = _gather3(
        tab_in, tab_out, blk_in, rem_in, blk_out, rem_out, blk_nz, rem_nz)
    return (o_in, o_out, o_nz.reshape(BATCH, N_SAMPLES, N_EMBED))
